# baseline (speedup 1.0000x reference)
import functools

import jax
import jax.numpy as jnp
from jax import lax
from jax.experimental import pallas as pl
from jax.experimental.pallas import tpu as pltpu
from jax.experimental.pallas import tpu_sc as plsc

T = 2048
D = 1024
E = 8
FF = 1024
EPS = 1e-5

R = 256
NBLK = 24
NPAD = NBLK * R
NW = 32
TPW = T // NW


def _cumsum_lanes(m):
    n = m.shape[-1]
    s = 1
    while s < n:
        shifted = jnp.pad(m, ((0, 0), (s, 0)))[:, :n]
        m = m + shifted
        s *= 2
    return m


def _stage1_body(x_ref, gamma_ref, wr_ref, xnb_ref, d0_ref, d1_ref,
                 w0_ref, w1_ref, bexp_ref, bval_ref):
    x = x_ref[...]
    var = jnp.mean(x * x, axis=-1, keepdims=True)
    xn = x * lax.rsqrt(var + EPS) * gamma_ref[...][None, :]
    xnb_ref[...] = xn

    logits = jnp.dot(xn, wr_ref[...], preferred_element_type=jnp.float32)
    scores = jax.nn.sigmoid(logits)
    i1 = jnp.argmax(scores, axis=-1)
    v1 = jnp.max(scores, axis=-1, keepdims=True)
    cols = lax.broadcasted_iota(jnp.int32, scores.shape, 1)
    masked = jnp.where(cols == i1[:, None], -jnp.inf, scores)
    i2 = jnp.argmax(masked, axis=-1)
    v2 = jnp.max(masked, axis=-1, keepdims=True)
    den = v1 + v2 + 1e-20
    w0_ref[...] = jnp.broadcast_to(v1 / den, (T, 16))
    w1_ref[...] = jnp.broadcast_to(v2 / den, (T, 16))

    erow = lax.broadcasted_iota(jnp.int32, (E, T), 0)
    oh1 = (erow == i1[None, :]).astype(jnp.float32)
    oh2 = (erow == i2[None, :]).astype(jnp.float32)
    m = oh1 + oh2
    cum = _cumsum_lanes(m)
    excl = cum - m
    counts = cum[:, T - 1][None, :]
    nblk_e = (counts.astype(jnp.int32) + (R - 1)) // R
    ends = _cumsum_lanes(nblk_e)
    baseblk = ends - nblk_e
    base_rows = (baseblk * R).astype(jnp.float32)
    base_col = base_rows.reshape(E, 1)
    d0 = jnp.sum(oh1 * (base_col + excl), axis=0)
    d1 = jnp.sum(oh2 * (base_col + excl), axis=0)
    d0_ref[...] = d0.astype(jnp.int32)
    d1_ref[...] = d1.astype(jnp.int32)

    brow = lax.broadcasted_iota(jnp.int32, (NBLK, E), 0)
    ends_b = jnp.broadcast_to(ends, (NBLK, E))
    bexp = jnp.sum((brow >= ends_b).astype(jnp.int32), axis=1)
    bval_ref[...] = (bexp < E).astype(jnp.int32)
    bexp_ref[...] = jnp.minimum(bexp, E - 1)


def _stage1(x, gamma, W_r):
    return pl.pallas_call(
        _stage1_body,
        grid=(1,),
        in_specs=[
            pl.BlockSpec((T, D), lambda i: (0, 0)),
            pl.BlockSpec((D,), lambda i: (0,)),
            pl.BlockSpec((D, E), lambda i: (0, 0)),
        ],
        out_specs=[
            pl.BlockSpec((T, D), lambda i: (0, 0)),
            pl.BlockSpec((T,), lambda i: (0,)),
            pl.BlockSpec((T,), lambda i: (0,)),
            pl.BlockSpec((T, 16), lambda i: (0, 0)),
            pl.BlockSpec((T, 16), lambda i: (0, 0)),
            pl.BlockSpec((NBLK,), lambda i: (0,)),
            pl.BlockSpec((NBLK,), lambda i: (0,)),
        ],
        out_shape=[
            jax.ShapeDtypeStruct((T, D), jnp.float32),
            jax.ShapeDtypeStruct((T,), jnp.int32),
            jax.ShapeDtypeStruct((T,), jnp.int32),
            jax.ShapeDtypeStruct((T, 16), jnp.float32),
            jax.ShapeDtypeStruct((T, 16), jnp.float32),
            jax.ShapeDtypeStruct((NBLK,), jnp.int32),
            jax.ShapeDtypeStruct((NBLK,), jnp.int32),
        ],
    )(x, gamma, W_r)


def _sc_scatter_body(xnb_hbm, d0_hbm, d1_hbm, xs_hbm, i0_v, i1_v, rows_v, sem):
    wid = lax.axis_index("s") * 2 + lax.axis_index("c")
    base = wid * TPW
    c0 = pltpu.async_copy(d0_hbm.at[pl.ds(base, TPW)], i0_v, sem)
    c1 = pltpu.async_copy(d1_hbm.at[pl.ds(base, TPW)], i1_v, sem)
    c2 = pltpu.async_copy(xnb_hbm.at[pl.ds(base, TPW)], rows_v, sem)
    c0.wait()
    c1.wait()
    c2.wait()
    cp0 = pltpu.async_copy(rows_v, xs_hbm.at[i0_v], sem)
    cp1 = pltpu.async_copy(rows_v, xs_hbm.at[i1_v], sem)
    cp0.wait()
    cp1.wait()


_CQ = 16
_NQ = TPW // _CQ


def _sc_combine_body(ys_hbm, x_hbm, d0_hbm, d1_hbm, w0_hbm, w1_hbm, out_hbm,
                     i0_v, i1_v, w0_v, w1_v, x_v, r0_v, r1_v, sg, so):
    wid = lax.axis_index("s") * 2 + lax.axis_index("c")
    fills = [None, None]
    outs = [None, None]
    for q in range(_NQ + 1):
        s = q % 2
        if q < _NQ:
            base = wid * TPW + q * _CQ
            if outs[s] is not None:
                outs[s].wait()
                outs[s] = None
            pltpu.sync_copy(d0_hbm.at[pl.ds(base, _CQ)], i0_v.at[s])
            pltpu.sync_copy(d1_hbm.at[pl.ds(base, _CQ)], i1_v.at[s])
            fills[s] = [
                pltpu.async_copy(ys_hbm.at[i0_v.at[s]], r0_v.at[s], sg),
                pltpu.async_copy(ys_hbm.at[i1_v.at[s]], r1_v.at[s], sg),
                pltpu.async_copy(w0_hbm.at[pl.ds(base, _CQ)], w0_v.at[s], sg),
                pltpu.async_copy(w1_hbm.at[pl.ds(base, _CQ)], w1_v.at[s], sg),
                pltpu.async_copy(x_hbm.at[pl.ds(base, _CQ)], x_v.at[s], sg),
            ]
        if q > 0:
            sp = (q - 1) % 2
            for cp in fills[sp]:
                cp.wait()

            def body(i, carry, sp=sp):
                w0 = w0_v[sp, i, :]
                w1 = w1_v[sp, i, :]
                for j in range(D // 16):
                    sl = pl.ds(j * 16, 16)
                    x_v[sp, i, sl] = (x_v[sp, i, sl] + w0 * r0_v[sp, i, sl]
                                      + w1 * r1_v[sp, i, sl])
                return carry

            lax.fori_loop(0, _CQ, body, 0)
            obase = wid * TPW + (q - 1) * _CQ
            outs[sp] = pltpu.async_copy(
                x_v.at[sp], out_hbm.at[pl.ds(obase, _CQ)], so)
    for s in range(2):
        if outs[s] is not None:
            outs[s].wait()


@functools.lru_cache(maxsize=None)
def _sc_kernels():
    mesh = plsc.VectorSubcoreMesh(core_axis_name="c", subcore_axis_name="s")
    scatter = pl.kernel(
        _sc_scatter_body,
        out_type=jax.ShapeDtypeStruct((NPAD, D), jnp.float32),
        mesh=mesh,
        scratch_types=[
            pltpu.VMEM((TPW,), jnp.int32),
            pltpu.VMEM((TPW,), jnp.int32),
            pltpu.VMEM((TPW, D), jnp.float32),
            pltpu.SemaphoreType.DMA,
        ],
    )
    combine = pl.kernel(
        _sc_combine_body,
        out_type=jax.ShapeDtypeStruct((T, D), jnp.float32),
        mesh=mesh,
        scratch_types=[
            pltpu.VMEM((2, _CQ), jnp.int32),
            pltpu.VMEM((2, _CQ), jnp.int32),
            pltpu.VMEM((2, _CQ, 16), jnp.float32),
            pltpu.VMEM((2, _CQ, 16), jnp.float32),
            pltpu.VMEM((2, _CQ, D), jnp.float32),
            pltpu.VMEM((2, _CQ, D), jnp.float32),
            pltpu.VMEM((2, _CQ, D), jnp.float32),
            pltpu.SemaphoreType.DMA,
            pltpu.SemaphoreType.DMA,
        ],
    )
    return scatter, combine


FH = FF // 2


def _stage2_body(bexp_ref, bval_ref, xs_ref, wg_ref, wu_ref, wd_ref, ys_ref,
                 wguc_ref, wdc_ref):
    b = pl.program_id(0)
    j = pl.program_id(1)

    @pl.when(bval_ref[b] == 1)
    def _():
        wguc_ref[:, :FH] = wg_ref[0].astype(jnp.bfloat16)
        wguc_ref[:, FH:] = wu_ref[0].astype(jnp.bfloat16)
        wdc_ref[...] = wd_ref[0].astype(jnp.bfloat16)
        xb = xs_ref[...].astype(jnp.bfloat16)
        gu = jnp.dot(xb, wguc_ref[...], preferred_element_type=jnp.float32)
        g = gu[:, :FH]
        u = gu[:, FH:]
        h = (g * jax.nn.sigmoid(g) * u).astype(jnp.bfloat16)
        y = jnp.dot(h, wdc_ref[...], preferred_element_type=jnp.float32)

        @pl.when(j == 0)
        def _():
            ys_ref[...] = y

        @pl.when(j == 1)
        def _():
            ys_ref[...] += y


def _stage2(bexp, bval, xs, wg_b, wu_b, wd_b):
    grid_spec = pltpu.PrefetchScalarGridSpec(
        num_scalar_prefetch=2,
        grid=(NBLK, 2),
        in_specs=[
            pl.BlockSpec((R, D),
                         lambda b, j, be, bv: (jnp.where(bv[b] == 1, b, 0), 0)),
            pl.BlockSpec((1, D, FH), lambda b, j, be, bv: (be[b], 0, j)),
            pl.BlockSpec((1, D, FH), lambda b, j, be, bv: (be[b], 0, j)),
            pl.BlockSpec((1, FH, D), lambda b, j, be, bv: (be[b], j, 0)),
        ],
        out_specs=pl.BlockSpec((R, D), lambda b, j, be, bv: (b, 0)),
        scratch_shapes=[
            pltpu.VMEM((D, FF), jnp.bfloat16),
            pltpu.VMEM((FH, D), jnp.bfloat16),
        ],
    )
    return pl.pallas_call(
        _stage2_body,
        grid_spec=grid_spec,
        out_shape=jax.ShapeDtypeStruct((NPAD, D), jnp.float32),
    )(bexp, bval, xs, wg_b, wu_b, wd_b)


def kernel(x, gamma, W_r, Wg, Wu, Wd):
    xnb, d0, d1, w0, w1, bexp, bval = _stage1(x, gamma, W_r)
    _sc_scatter, _sc_combine = _sc_kernels()
    xs = _sc_scatter(xnb, d0, d1)
    ys = _stage2(bexp, bval, xs, Wg, Wu, Wd)
    return _sc_combine(ys, x, d0, d1, w0, w1)


# speedup vs baseline: 1.2628x; 1.2628x over previous
import functools

import jax
import jax.numpy as jnp
from jax import lax
from jax.experimental import pallas as pl
from jax.experimental.pallas import tpu as pltpu
from jax.experimental.pallas import tpu_sc as plsc

T = 2048
D = 1024
E = 8
FF = 1024
EPS = 1e-5

R = 256
NBLK = 24
NPAD = NBLK * R
NW = 32
TPW = T // NW


def _cumsum_lanes(m):
    n = m.shape[-1]
    s = 1
    while s < n:
        shifted = jnp.pad(m, ((0, 0), (s, 0)))[:, :n]
        m = m + shifted
        s *= 2
    return m


def _stage1_body(x_ref, gamma_ref, wr_ref, xnb_ref, d0_ref, d1_ref,
                 w0_ref, w1_ref, bexp_ref, bval_ref):
    x = x_ref[...]
    var = jnp.mean(x * x, axis=-1, keepdims=True)
    xn = x * lax.rsqrt(var + EPS) * gamma_ref[...][None, :]
    xnb_ref[...] = xn

    logits = jnp.dot(xn, wr_ref[...], preferred_element_type=jnp.float32)
    scores = jax.nn.sigmoid(logits)
    i1 = jnp.argmax(scores, axis=-1)
    v1 = jnp.max(scores, axis=-1, keepdims=True)
    cols = lax.broadcasted_iota(jnp.int32, scores.shape, 1)
    masked = jnp.where(cols == i1[:, None], -jnp.inf, scores)
    i2 = jnp.argmax(masked, axis=-1)
    v2 = jnp.max(masked, axis=-1, keepdims=True)
    den = v1 + v2 + 1e-20
    w0_ref[...] = jnp.broadcast_to(v1 / den, (T, 16))
    w1_ref[...] = jnp.broadcast_to(v2 / den, (T, 16))

    erow = lax.broadcasted_iota(jnp.int32, (E, T), 0)
    oh1 = (erow == i1[None, :]).astype(jnp.float32)
    oh2 = (erow == i2[None, :]).astype(jnp.float32)
    m = oh1 + oh2
    cum = _cumsum_lanes(m)
    excl = cum - m
    counts = cum[:, T - 1][None, :]
    nblk_e = (counts.astype(jnp.int32) + (R - 1)) // R
    ends = _cumsum_lanes(nblk_e)
    baseblk = ends - nblk_e
    base_rows = (baseblk * R).astype(jnp.float32)
    base_col = base_rows.reshape(E, 1)
    d0 = jnp.sum(oh1 * (base_col + excl), axis=0)
    d1 = jnp.sum(oh2 * (base_col + excl), axis=0)
    d0_ref[...] = d0.astype(jnp.int32)
    d1_ref[...] = d1.astype(jnp.int32)

    brow = lax.broadcasted_iota(jnp.int32, (NBLK, E), 0)
    ends_b = jnp.broadcast_to(ends, (NBLK, E))
    bexp = jnp.sum((brow >= ends_b).astype(jnp.int32), axis=1)
    bval_ref[...] = (bexp < E).astype(jnp.int32)
    bexp_ref[...] = jnp.minimum(bexp, E - 1)


def _stage1(x, gamma, W_r):
    return pl.pallas_call(
        _stage1_body,
        grid=(1,),
        in_specs=[
            pl.BlockSpec((T, D), lambda i: (0, 0)),
            pl.BlockSpec((D,), lambda i: (0,)),
            pl.BlockSpec((D, E), lambda i: (0, 0)),
        ],
        out_specs=[
            pl.BlockSpec((T, D), lambda i: (0, 0)),
            pl.BlockSpec((T,), lambda i: (0,)),
            pl.BlockSpec((T,), lambda i: (0,)),
            pl.BlockSpec((T, 16), lambda i: (0, 0)),
            pl.BlockSpec((T, 16), lambda i: (0, 0)),
            pl.BlockSpec((NBLK,), lambda i: (0,)),
            pl.BlockSpec((NBLK,), lambda i: (0,)),
        ],
        out_shape=[
            jax.ShapeDtypeStruct((T, D), jnp.float32),
            jax.ShapeDtypeStruct((T,), jnp.int32),
            jax.ShapeDtypeStruct((T,), jnp.int32),
            jax.ShapeDtypeStruct((T, 16), jnp.float32),
            jax.ShapeDtypeStruct((T, 16), jnp.float32),
            jax.ShapeDtypeStruct((NBLK,), jnp.int32),
            jax.ShapeDtypeStruct((NBLK,), jnp.int32),
        ],
    )(x, gamma, W_r)


def _sc_scatter_body(xnb_hbm, d0_hbm, d1_hbm, xs_hbm, i0_v, i1_v, rows_v, sem):
    wid = lax.axis_index("s") * 2 + lax.axis_index("c")
    base = wid * TPW
    c0 = pltpu.async_copy(d0_hbm.at[pl.ds(base, TPW)], i0_v, sem)
    c1 = pltpu.async_copy(d1_hbm.at[pl.ds(base, TPW)], i1_v, sem)
    c2 = pltpu.async_copy(xnb_hbm.at[pl.ds(base, TPW)], rows_v, sem)
    c0.wait()
    c1.wait()
    c2.wait()
    cp0 = pltpu.async_copy(rows_v, xs_hbm.at[i0_v], sem)
    cp1 = pltpu.async_copy(rows_v, xs_hbm.at[i1_v], sem)
    cp0.wait()
    cp1.wait()


_CQ = 16
_NQ = TPW // _CQ


def _sc_combine_body(ys_hbm, x_hbm, d0_hbm, d1_hbm, w0_hbm, w1_hbm, out_hbm,
                     i0_v, i1_v, w0_v, w1_v, x_v, r0_v, r1_v, sg, so):
    wid = lax.axis_index("s") * 2 + lax.axis_index("c")
    fills = [None, None]
    outs = [None, None]
    for q in range(_NQ + 1):
        s = q % 2
        if q < _NQ:
            base = wid * TPW + q * _CQ
            if outs[s] is not None:
                outs[s].wait()
                outs[s] = None
            pltpu.sync_copy(d0_hbm.at[pl.ds(base, _CQ)], i0_v.at[s])
            pltpu.sync_copy(d1_hbm.at[pl.ds(base, _CQ)], i1_v.at[s])
            fills[s] = [
                pltpu.async_copy(ys_hbm.at[i0_v.at[s]], r0_v.at[s], sg),
                pltpu.async_copy(ys_hbm.at[i1_v.at[s]], r1_v.at[s], sg),
                pltpu.async_copy(w0_hbm.at[pl.ds(base, _CQ)], w0_v.at[s], sg),
                pltpu.async_copy(w1_hbm.at[pl.ds(base, _CQ)], w1_v.at[s], sg),
                pltpu.async_copy(x_hbm.at[pl.ds(base, _CQ)], x_v.at[s], sg),
            ]
        if q > 0:
            sp = (q - 1) % 2
            for cp in fills[sp]:
                cp.wait()

            def body(i, carry, sp=sp):
                w0 = w0_v[sp, i, :]
                w1 = w1_v[sp, i, :]
                for j in range(D // 16):
                    sl = pl.ds(j * 16, 16)
                    x_v[sp, i, sl] = (x_v[sp, i, sl] + w0 * r0_v[sp, i, sl]
                                      + w1 * r1_v[sp, i, sl])
                return carry

            lax.fori_loop(0, _CQ, body, 0)
            obase = wid * TPW + (q - 1) * _CQ
            outs[sp] = pltpu.async_copy(
                x_v.at[sp], out_hbm.at[pl.ds(obase, _CQ)], so)
    for s in range(2):
        if outs[s] is not None:
            outs[s].wait()


@functools.lru_cache(maxsize=None)
def _sc_kernels():
    mesh = plsc.VectorSubcoreMesh(core_axis_name="c", subcore_axis_name="s")
    scatter = pl.kernel(
        _sc_scatter_body,
        out_type=jax.ShapeDtypeStruct((NPAD, D), jnp.float32),
        mesh=mesh,
        scratch_types=[
            pltpu.VMEM((TPW,), jnp.int32),
            pltpu.VMEM((TPW,), jnp.int32),
            pltpu.VMEM((TPW, D), jnp.float32),
            pltpu.SemaphoreType.DMA,
        ],
    )
    combine = pl.kernel(
        _sc_combine_body,
        out_type=jax.ShapeDtypeStruct((T, D), jnp.float32),
        mesh=mesh,
        scratch_types=[
            pltpu.VMEM((2, _CQ), jnp.int32),
            pltpu.VMEM((2, _CQ), jnp.int32),
            pltpu.VMEM((2, _CQ, 16), jnp.float32),
            pltpu.VMEM((2, _CQ, 16), jnp.float32),
            pltpu.VMEM((2, _CQ, D), jnp.float32),
            pltpu.VMEM((2, _CQ, D), jnp.float32),
            pltpu.VMEM((2, _CQ, D), jnp.float32),
            pltpu.SemaphoreType.DMA,
            pltpu.SemaphoreType.DMA,
        ],
    )
    return scatter, combine


def _stage2_body(bexp_ref, bval_ref, xs_ref, wg_ref, wu_ref, wd_ref, ys_ref,
                 wguc_ref, wdc_ref):
    b = pl.program_id(0)
    prev = jnp.maximum(b - 1, 0)
    new_expert = jnp.logical_or(b == 0, bexp_ref[b] != bexp_ref[prev])

    @pl.when(jnp.logical_and(bval_ref[b] == 1, new_expert))
    def _():
        wguc_ref[:, :FF] = wg_ref[0].astype(jnp.bfloat16)
        wguc_ref[:, FF:] = wu_ref[0].astype(jnp.bfloat16)
        wdc_ref[...] = wd_ref[0].astype(jnp.bfloat16)

    @pl.when(bval_ref[b] == 1)
    def _():
        xb = xs_ref[...].astype(jnp.bfloat16)
        gu = jnp.dot(xb, wguc_ref[...], preferred_element_type=jnp.float32)
        g = gu[:, :FF]
        u = gu[:, FF:]
        h = (g * jax.nn.sigmoid(g) * u).astype(jnp.bfloat16)
        y = jnp.dot(h, wdc_ref[...], preferred_element_type=jnp.float32)
        ys_ref[...] = y


def _stage2(bexp, bval, xs, wg_b, wu_b, wd_b):
    grid_spec = pltpu.PrefetchScalarGridSpec(
        num_scalar_prefetch=2,
        grid=(NBLK,),
        in_specs=[
            pl.BlockSpec((R, D),
                         lambda b, be, bv: (jnp.where(bv[b] == 1, b, NBLK - 1), 0)),
            pl.BlockSpec((1, D, FF), lambda b, be, bv: (be[b], 0, 0)),
            pl.BlockSpec((1, D, FF), lambda b, be, bv: (be[b], 0, 0)),
            pl.BlockSpec((1, FF, D), lambda b, be, bv: (be[b], 0, 0)),
        ],
        out_specs=pl.BlockSpec((R, D), lambda b, be, bv: (b, 0)),
        scratch_shapes=[
            pltpu.VMEM((D, 2 * FF), jnp.bfloat16),
            pltpu.VMEM((FF, D), jnp.bfloat16),
        ],
    )
    return pl.pallas_call(
        _stage2_body,
        grid_spec=grid_spec,
        out_shape=jax.ShapeDtypeStruct((NPAD, D), jnp.float32),
    )(bexp, bval, xs, wg_b, wu_b, wd_b)


def kernel(x, gamma, W_r, Wg, Wu, Wd):
    xnb, d0, d1, w0, w1, bexp, bval = _stage1(x, gamma, W_r)
    _sc_scatter, _sc_combine = _sc_kernels()
    xs = _sc_scatter(xnb, d0, d1)
    ys = _stage2(bexp, bval, xs, Wg, Wu, Wd)
    return _sc_combine(ys, x, d0, d1, w0, w1)


# speedup vs baseline: 1.4567x; 1.1535x over previous
import functools

import jax
import jax.numpy as jnp
from jax import lax
from jax.experimental import pallas as pl
from jax.experimental.pallas import tpu as pltpu
from jax.experimental.pallas import tpu_sc as plsc

T = 2048
D = 1024
E = 8
FF = 1024
EPS = 1e-5

R = 256
NBLK = 24
NPAD = NBLK * R
NW = 32
TPW = T // NW


def _cumsum_lanes(m):
    n = m.shape[-1]
    s = 1
    while s < n:
        shifted = jnp.pad(m, ((0, 0), (s, 0)))[:, :n]
        m = m + shifted
        s *= 2
    return m


def _stage1_body(x_ref, gamma_ref, wr_ref, xnb_ref, d0_ref, d1_ref,
                 w0_ref, w1_ref, bexp_ref, bval_ref, fstb_ref, nxe_ref,
                 hnx_ref, sgo_ref):
    x = x_ref[...]
    var = jnp.mean(x * x, axis=-1, keepdims=True)
    xn = x * lax.rsqrt(var + EPS) * gamma_ref[...][None, :]
    xnb_ref[...] = xn

    logits = jnp.dot(xn, wr_ref[...], preferred_element_type=jnp.float32)
    scores = jax.nn.sigmoid(logits)
    i1 = jnp.argmax(scores, axis=-1)
    v1 = jnp.max(scores, axis=-1, keepdims=True)
    cols = lax.broadcasted_iota(jnp.int32, scores.shape, 1)
    masked = jnp.where(cols == i1[:, None], -jnp.inf, scores)
    i2 = jnp.argmax(masked, axis=-1)
    v2 = jnp.max(masked, axis=-1, keepdims=True)
    den = v1 + v2 + 1e-20
    w0_ref[...] = jnp.broadcast_to(v1 / den, (T, 16))
    w1_ref[...] = jnp.broadcast_to(v2 / den, (T, 16))

    erow = lax.broadcasted_iota(jnp.int32, (E, T), 0)
    oh1 = (erow == i1[None, :]).astype(jnp.float32)
    oh2 = (erow == i2[None, :]).astype(jnp.float32)
    m = oh1 + oh2
    cum = _cumsum_lanes(m)
    excl = cum - m
    counts = cum[:, T - 1][None, :]
    nblk_e = (counts.astype(jnp.int32) + (R - 1)) // R
    ends = _cumsum_lanes(nblk_e)
    baseblk = ends - nblk_e
    base_rows = (baseblk * R).astype(jnp.float32)
    base_col = base_rows.reshape(E, 1)
    d0 = jnp.sum(oh1 * (base_col + excl), axis=0)
    d1 = jnp.sum(oh2 * (base_col + excl), axis=0)
    d0_ref[...] = d0.astype(jnp.int32)
    d1_ref[...] = d1.astype(jnp.int32)

    brow = lax.broadcasted_iota(jnp.int32, (NBLK, E), 0)
    ecol = lax.broadcasted_iota(jnp.int32, (NBLK, E), 1)
    ends_b = jnp.broadcast_to(ends, (NBLK, E))
    bexp = jnp.sum((brow >= ends_b).astype(jnp.int32), axis=1)
    bexp_c = jnp.minimum(bexp, E - 1)
    bval_ref[...] = (bexp < E).astype(jnp.int32)
    bexp_ref[...] = bexp_c

    ohb = (bexp_c[:, None] == ecol).astype(jnp.int32)
    baseblk_b = jnp.broadcast_to(baseblk, (NBLK, E))
    fstb_ref[...] = jnp.sum(ohb * baseblk_b, axis=1)
    eob = jnp.sum(ohb * ends_b, axis=1)
    nxe_raw = jnp.sum((eob[:, None] >= ends_b).astype(jnp.int32), axis=1)
    nxe_ref[...] = jnp.minimum(nxe_raw, E - 1)
    hnx_ref[...] = (eob < ends_b[:, E - 1]).astype(jnp.int32)
    pres_b = jnp.broadcast_to((nblk_e > 0).astype(jnp.int32), (NBLK, E))
    sgo_ref[...] = jnp.sum(pres_b * (ecol < bexp_c[:, None]).astype(jnp.int32),
                           axis=1)


def _stage1(x, gamma, W_r):
    return pl.pallas_call(
        _stage1_body,
        grid=(1,),
        in_specs=[
            pl.BlockSpec((T, D), lambda i: (0, 0)),
            pl.BlockSpec((D,), lambda i: (0,)),
            pl.BlockSpec((D, E), lambda i: (0, 0)),
        ],
        out_specs=[
            pl.BlockSpec((T, D), lambda i: (0, 0)),
            pl.BlockSpec((T,), lambda i: (0,)),
            pl.BlockSpec((T,), lambda i: (0,)),
            pl.BlockSpec((T, 16), lambda i: (0, 0)),
            pl.BlockSpec((T, 16), lambda i: (0, 0)),
            pl.BlockSpec((NBLK,), lambda i: (0,)),
            pl.BlockSpec((NBLK,), lambda i: (0,)),
            pl.BlockSpec((NBLK,), lambda i: (0,)),
            pl.BlockSpec((NBLK,), lambda i: (0,)),
            pl.BlockSpec((NBLK,), lambda i: (0,)),
            pl.BlockSpec((NBLK,), lambda i: (0,)),
        ],
        out_shape=[
            jax.ShapeDtypeStruct((T, D), jnp.float32),
            jax.ShapeDtypeStruct((T,), jnp.int32),
            jax.ShapeDtypeStruct((T,), jnp.int32),
            jax.ShapeDtypeStruct((T, 16), jnp.float32),
            jax.ShapeDtypeStruct((T, 16), jnp.float32),
            jax.ShapeDtypeStruct((NBLK,), jnp.int32),
            jax.ShapeDtypeStruct((NBLK,), jnp.int32),
            jax.ShapeDtypeStruct((NBLK,), jnp.int32),
            jax.ShapeDtypeStruct((NBLK,), jnp.int32),
            jax.ShapeDtypeStruct((NBLK,), jnp.int32),
            jax.ShapeDtypeStruct((NBLK,), jnp.int32),
        ],
    )(x, gamma, W_r)


def _sc_scatter_body(xnb_hbm, d0_hbm, d1_hbm, xs_hbm, i0_v, i1_v, rows_v, sem):
    wid = lax.axis_index("s") * 2 + lax.axis_index("c")
    base = wid * TPW
    c0 = pltpu.async_copy(d0_hbm.at[pl.ds(base, TPW)], i0_v, sem)
    c1 = pltpu.async_copy(d1_hbm.at[pl.ds(base, TPW)], i1_v, sem)
    c2 = pltpu.async_copy(xnb_hbm.at[pl.ds(base, TPW)], rows_v, sem)
    c0.wait()
    c1.wait()
    c2.wait()
    cp0 = pltpu.async_copy(rows_v, xs_hbm.at[i0_v], sem)
    cp1 = pltpu.async_copy(rows_v, xs_hbm.at[i1_v], sem)
    cp0.wait()
    cp1.wait()


_CQ = 16
_NQ = TPW // _CQ


def _sc_combine_body(ys_hbm, x_hbm, d0_hbm, d1_hbm, w0_hbm, w1_hbm, out_hbm,
                     i0_v, i1_v, w0_v, w1_v, x_v, r0_v, r1_v, sg, so):
    wid = lax.axis_index("s") * 2 + lax.axis_index("c")
    fills = [None, None]
    outs = [None, None]
    for q in range(_NQ + 1):
        s = q % 2
        if q < _NQ:
            base = wid * TPW + q * _CQ
            if outs[s] is not None:
                outs[s].wait()
                outs[s] = None
            pltpu.sync_copy(d0_hbm.at[pl.ds(base, _CQ)], i0_v.at[s])
            pltpu.sync_copy(d1_hbm.at[pl.ds(base, _CQ)], i1_v.at[s])
            fills[s] = [
                pltpu.async_copy(ys_hbm.at[i0_v.at[s]], r0_v.at[s], sg),
                pltpu.async_copy(ys_hbm.at[i1_v.at[s]], r1_v.at[s], sg),
                pltpu.async_copy(w0_hbm.at[pl.ds(base, _CQ)], w0_v.at[s], sg),
                pltpu.async_copy(w1_hbm.at[pl.ds(base, _CQ)], w1_v.at[s], sg),
                pltpu.async_copy(x_hbm.at[pl.ds(base, _CQ)], x_v.at[s], sg),
            ]
        if q > 0:
            sp = (q - 1) % 2
            for cp in fills[sp]:
                cp.wait()

            def body(i, carry, sp=sp):
                w0 = w0_v[sp, i, :]
                w1 = w1_v[sp, i, :]
                for j in range(D // 16):
                    sl = pl.ds(j * 16, 16)
                    x_v[sp, i, sl] = (x_v[sp, i, sl] + w0 * r0_v[sp, i, sl]
                                      + w1 * r1_v[sp, i, sl])
                return carry

            lax.fori_loop(0, _CQ, body, 0)
            obase = wid * TPW + (q - 1) * _CQ
            outs[sp] = pltpu.async_copy(
                x_v.at[sp], out_hbm.at[pl.ds(obase, _CQ)], so)
    for s in range(2):
        if outs[s] is not None:
            outs[s].wait()


@functools.lru_cache(maxsize=None)
def _sc_kernels():
    mesh = plsc.VectorSubcoreMesh(core_axis_name="c", subcore_axis_name="s")
    scatter = pl.kernel(
        _sc_scatter_body,
        out_type=jax.ShapeDtypeStruct((NPAD, D), jnp.float32),
        mesh=mesh,
        scratch_types=[
            pltpu.VMEM((TPW,), jnp.int32),
            pltpu.VMEM((TPW,), jnp.int32),
            pltpu.VMEM((TPW, D), jnp.float32),
            pltpu.SemaphoreType.DMA,
        ],
    )
    combine = pl.kernel(
        _sc_combine_body,
        out_type=jax.ShapeDtypeStruct((T, D), jnp.float32),
        mesh=mesh,
        scratch_types=[
            pltpu.VMEM((2, _CQ), jnp.int32),
            pltpu.VMEM((2, _CQ), jnp.int32),
            pltpu.VMEM((2, _CQ, 16), jnp.float32),
            pltpu.VMEM((2, _CQ, 16), jnp.float32),
            pltpu.VMEM((2, _CQ, D), jnp.float32),
            pltpu.VMEM((2, _CQ, D), jnp.float32),
            pltpu.VMEM((2, _CQ, D), jnp.float32),
            pltpu.SemaphoreType.DMA,
            pltpu.SemaphoreType.DMA,
        ],
    )
    return scatter, combine


def _stage2_body(bexp_ref, bval_ref, fstb_ref, nxe_ref, hnx_ref, sgo_ref,
                 xs_ref, wg_any, wu_any, wd_any, ys_ref,
                 wbuf_ref, wguc_ref, wdc_ref, sem):
    b = pl.program_id(0)
    e_cur = bexp_ref[b]
    slot = lax.rem(sgo_ref[b], 2)
    nslot = 1 - slot
    valid = bval_ref[b] == 1
    first = jnp.logical_and(valid, b == fstb_ref[b])

    def copies(e, s):
        return [
            pltpu.make_async_copy(wg_any.at[e], wbuf_ref.at[s, 0], sem),
            pltpu.make_async_copy(wu_any.at[e], wbuf_ref.at[s, 1], sem),
            pltpu.make_async_copy(wd_any.at[e], wbuf_ref.at[s, 2], sem),
        ]

    @pl.when(b == 0)
    def _():
        for c in copies(e_cur, slot):
            c.start()

    @pl.when(first)
    def _():
        for c in copies(e_cur, slot):
            c.wait()

        @pl.when(hnx_ref[b] == 1)
        def _():
            for c in copies(nxe_ref[b], nslot):
                c.start()

        wguc_ref[:, :FF] = wbuf_ref[slot, 0].astype(jnp.bfloat16)
        wguc_ref[:, FF:] = wbuf_ref[slot, 1].astype(jnp.bfloat16)
        wdc_ref[...] = wbuf_ref[slot, 2].astype(jnp.bfloat16)

    @pl.when(valid)
    def _():
        xb = xs_ref[...].astype(jnp.bfloat16)
        gu = jnp.dot(xb, wguc_ref[...], preferred_element_type=jnp.float32)
        g = gu[:, :FF]
        u = gu[:, FF:]
        h = (g * jax.nn.sigmoid(g) * u).astype(jnp.bfloat16)
        y = jnp.dot(h, wdc_ref[...], preferred_element_type=jnp.float32)
        ys_ref[...] = y


def _stage2(bexp, bval, fstb, nxe, hnx, sgo, xs, Wg, Wu, Wd):
    grid_spec = pltpu.PrefetchScalarGridSpec(
        num_scalar_prefetch=6,
        grid=(NBLK,),
        in_specs=[
            pl.BlockSpec(
                (R, D),
                lambda b, be, bv, fb, nx, hn, sg:
                    (jnp.where(bv[b] == 1, b, NBLK - 1), 0)),
            pl.BlockSpec(memory_space=pl.ANY),
            pl.BlockSpec(memory_space=pl.ANY),
            pl.BlockSpec(memory_space=pl.ANY),
        ],
        out_specs=pl.BlockSpec((R, D),
                               lambda b, be, bv, fb, nx, hn, sg: (b, 0)),
        scratch_shapes=[
            pltpu.VMEM((2, 3, D, FF), jnp.float32),
            pltpu.VMEM((D, 2 * FF), jnp.bfloat16),
            pltpu.VMEM((FF, D), jnp.bfloat16),
            pltpu.SemaphoreType.DMA,
        ],
    )
    return pl.pallas_call(
        _stage2_body,
        grid_spec=grid_spec,
        out_shape=jax.ShapeDtypeStruct((NPAD, D), jnp.float32),
    )(bexp, bval, fstb, nxe, hnx, sgo, xs, Wg, Wu, Wd)


def kernel(x, gamma, W_r, Wg, Wu, Wd):
    (xnb, d0, d1, w0, w1, bexp, bval,
     fstb, nxe, hnx, sgo) = _stage1(x, gamma, W_r)
    _sc_scatter, _sc_combine = _sc_kernels()
    xs = _sc_scatter(xnb, d0, d1)
    ys = _stage2(bexp, bval, fstb, nxe, hnx, sgo, xs, Wg, Wu, Wd)
    return _sc_combine(ys, x, d0, d1, w0, w1)


# speedup vs baseline: 1.4594x; 1.0019x over previous
import functools

import jax
import jax.numpy as jnp
from jax import lax
from jax.experimental import pallas as pl
from jax.experimental.pallas import tpu as pltpu
from jax.experimental.pallas import tpu_sc as plsc

T = 2048
D = 1024
E = 8
FF = 1024
EPS = 1e-5

R = 256
NBLK = 24
NPAD = NBLK * R
NW = 32
TPW = T // NW


def _cumsum_lanes(m):
    n = m.shape[-1]
    s = 1
    while s < n:
        shifted = jnp.pad(m, ((0, 0), (s, 0)))[:, :n]
        m = m + shifted
        s *= 2
    return m


def _stage1_body(x_ref, gamma_ref, wr_ref, xnb_ref, d0_ref, d1_ref,
                 w0_ref, w1_ref, bexp_ref, bval_ref, fstb_ref, nxe_ref,
                 hnx_ref, sgo_ref):
    x = x_ref[...]
    var = jnp.mean(x * x, axis=-1, keepdims=True)
    xn = x * lax.rsqrt(var + EPS) * gamma_ref[...][None, :]
    xnb_ref[...] = xn

    logits = jnp.dot(xn, wr_ref[...], preferred_element_type=jnp.float32)
    scores = jax.nn.sigmoid(logits)
    i1 = jnp.argmax(scores, axis=-1)
    v1 = jnp.max(scores, axis=-1, keepdims=True)
    cols = lax.broadcasted_iota(jnp.int32, scores.shape, 1)
    masked = jnp.where(cols == i1[:, None], -jnp.inf, scores)
    i2 = jnp.argmax(masked, axis=-1)
    v2 = jnp.max(masked, axis=-1, keepdims=True)
    den = v1 + v2 + 1e-20
    w0_ref[...] = jnp.broadcast_to(v1 / den, (T, 16))
    w1_ref[...] = jnp.broadcast_to(v2 / den, (T, 16))

    erow = lax.broadcasted_iota(jnp.int32, (E, T), 0)
    oh1 = (erow == i1[None, :]).astype(jnp.float32)
    oh2 = (erow == i2[None, :]).astype(jnp.float32)
    m = oh1 + oh2
    cum = _cumsum_lanes(m)
    excl = cum - m
    counts = cum[:, T - 1][None, :]
    nblk_e = (counts.astype(jnp.int32) + (R - 1)) // R
    ends = _cumsum_lanes(nblk_e)
    baseblk = ends - nblk_e
    base_rows = (baseblk * R).astype(jnp.float32)
    base_col = base_rows.reshape(E, 1)
    d0 = jnp.sum(oh1 * (base_col + excl), axis=0)
    d1 = jnp.sum(oh2 * (base_col + excl), axis=0)
    d0_ref[...] = d0.astype(jnp.int32)
    d1_ref[...] = d1.astype(jnp.int32)

    brow = lax.broadcasted_iota(jnp.int32, (NBLK, E), 0)
    ecol = lax.broadcasted_iota(jnp.int32, (NBLK, E), 1)
    ends_b = jnp.broadcast_to(ends, (NBLK, E))
    bexp = jnp.sum((brow >= ends_b).astype(jnp.int32), axis=1)
    bexp_c = jnp.minimum(bexp, E - 1)
    bval_ref[...] = (bexp < E).astype(jnp.int32)
    bexp_ref[...] = bexp_c

    ohb = (bexp_c[:, None] == ecol).astype(jnp.int32)
    baseblk_b = jnp.broadcast_to(baseblk, (NBLK, E))
    fstb_ref[...] = jnp.sum(ohb * baseblk_b, axis=1)
    eob = jnp.sum(ohb * ends_b, axis=1)
    nxe_raw = jnp.sum((eob[:, None] >= ends_b).astype(jnp.int32), axis=1)
    nxe_ref[...] = jnp.minimum(nxe_raw, E - 1)
    hnx_ref[...] = (eob < ends_b[:, E - 1]).astype(jnp.int32)
    pres_b = jnp.broadcast_to((nblk_e > 0).astype(jnp.int32), (NBLK, E))
    sgo_ref[...] = jnp.sum(pres_b * (ecol < bexp_c[:, None]).astype(jnp.int32),
                           axis=1)


def _stage1(x, gamma, W_r):
    return pl.pallas_call(
        _stage1_body,
        grid=(1,),
        in_specs=[
            pl.BlockSpec((T, D), lambda i: (0, 0)),
            pl.BlockSpec((D,), lambda i: (0,)),
            pl.BlockSpec((D, E), lambda i: (0, 0)),
        ],
        out_specs=[
            pl.BlockSpec((T, D), lambda i: (0, 0)),
            pl.BlockSpec((T,), lambda i: (0,)),
            pl.BlockSpec((T,), lambda i: (0,)),
            pl.BlockSpec((T, 16), lambda i: (0, 0)),
            pl.BlockSpec((T, 16), lambda i: (0, 0)),
            pl.BlockSpec((NBLK,), lambda i: (0,)),
            pl.BlockSpec((NBLK,), lambda i: (0,)),
            pl.BlockSpec((NBLK,), lambda i: (0,)),
            pl.BlockSpec((NBLK,), lambda i: (0,)),
            pl.BlockSpec((NBLK,), lambda i: (0,)),
            pl.BlockSpec((NBLK,), lambda i: (0,)),
        ],
        out_shape=[
            jax.ShapeDtypeStruct((T, D), jnp.float32),
            jax.ShapeDtypeStruct((T,), jnp.int32),
            jax.ShapeDtypeStruct((T,), jnp.int32),
            jax.ShapeDtypeStruct((T, 16), jnp.float32),
            jax.ShapeDtypeStruct((T, 16), jnp.float32),
            jax.ShapeDtypeStruct((NBLK,), jnp.int32),
            jax.ShapeDtypeStruct((NBLK,), jnp.int32),
            jax.ShapeDtypeStruct((NBLK,), jnp.int32),
            jax.ShapeDtypeStruct((NBLK,), jnp.int32),
            jax.ShapeDtypeStruct((NBLK,), jnp.int32),
            jax.ShapeDtypeStruct((NBLK,), jnp.int32),
        ],
    )(x, gamma, W_r)


_SH = TPW // 2


def _sc_scatter_body(xnb_hbm, d0_hbm, d1_hbm, xs_hbm, i0_v, i1_v, rows_v, sem):
    wid = lax.axis_index("s") * 2 + lax.axis_index("c")
    fills = []
    for h in range(2):
        base = wid * TPW + h * _SH
        fills.append([
            pltpu.async_copy(d0_hbm.at[pl.ds(base, _SH)], i0_v.at[h], sem),
            pltpu.async_copy(d1_hbm.at[pl.ds(base, _SH)], i1_v.at[h], sem),
            pltpu.async_copy(xnb_hbm.at[pl.ds(base, _SH)], rows_v.at[h], sem),
        ])
    scats = []
    for h in range(2):
        for c in fills[h]:
            c.wait()
        scats += [
            pltpu.async_copy(rows_v.at[h], xs_hbm.at[i0_v.at[h]], sem),
            pltpu.async_copy(rows_v.at[h], xs_hbm.at[i1_v.at[h]], sem),
        ]
    for c in scats:
        c.wait()


_CQ = 16
_NQ = TPW // _CQ


def _sc_combine_body(ys_hbm, x_hbm, d0_hbm, d1_hbm, w0_hbm, w1_hbm, out_hbm,
                     i0_v, i1_v, w0_v, w1_v, x_v, r0_v, r1_v, sg, so):
    wid = lax.axis_index("s") * 2 + lax.axis_index("c")
    fills = [None, None]
    outs = [None, None]
    for q in range(_NQ + 1):
        s = q % 2
        if q < _NQ:
            base = wid * TPW + q * _CQ
            if outs[s] is not None:
                outs[s].wait()
                outs[s] = None
            pltpu.sync_copy(d0_hbm.at[pl.ds(base, _CQ)], i0_v.at[s])
            pltpu.sync_copy(d1_hbm.at[pl.ds(base, _CQ)], i1_v.at[s])
            fills[s] = [
                pltpu.async_copy(ys_hbm.at[i0_v.at[s]], r0_v.at[s], sg),
                pltpu.async_copy(ys_hbm.at[i1_v.at[s]], r1_v.at[s], sg),
                pltpu.async_copy(w0_hbm.at[pl.ds(base, _CQ)], w0_v.at[s], sg),
                pltpu.async_copy(w1_hbm.at[pl.ds(base, _CQ)], w1_v.at[s], sg),
                pltpu.async_copy(x_hbm.at[pl.ds(base, _CQ)], x_v.at[s], sg),
            ]
        if q > 0:
            sp = (q - 1) % 2
            for cp in fills[sp]:
                cp.wait()

            def body(i, carry, sp=sp):
                w0 = w0_v[sp, i, :]
                w1 = w1_v[sp, i, :]
                for j in range(D // 16):
                    sl = pl.ds(j * 16, 16)
                    x_v[sp, i, sl] = (x_v[sp, i, sl] + w0 * r0_v[sp, i, sl]
                                      + w1 * r1_v[sp, i, sl])
                return carry

            lax.fori_loop(0, _CQ, body, 0)
            obase = wid * TPW + (q - 1) * _CQ
            outs[sp] = pltpu.async_copy(
                x_v.at[sp], out_hbm.at[pl.ds(obase, _CQ)], so)
    for s in range(2):
        if outs[s] is not None:
            outs[s].wait()


@functools.lru_cache(maxsize=None)
def _sc_kernels():
    mesh = plsc.VectorSubcoreMesh(core_axis_name="c", subcore_axis_name="s")
    scatter = pl.kernel(
        _sc_scatter_body,
        out_type=jax.ShapeDtypeStruct((NPAD, D), jnp.float32),
        mesh=mesh,
        scratch_types=[
            pltpu.VMEM((2, _SH), jnp.int32),
            pltpu.VMEM((2, _SH), jnp.int32),
            pltpu.VMEM((2, _SH, D), jnp.float32),
            pltpu.SemaphoreType.DMA,
        ],
    )
    combine = pl.kernel(
        _sc_combine_body,
        out_type=jax.ShapeDtypeStruct((T, D), jnp.float32),
        mesh=mesh,
        scratch_types=[
            pltpu.VMEM((2, _CQ), jnp.int32),
            pltpu.VMEM((2, _CQ), jnp.int32),
            pltpu.VMEM((2, _CQ, 16), jnp.float32),
            pltpu.VMEM((2, _CQ, 16), jnp.float32),
            pltpu.VMEM((2, _CQ, D), jnp.float32),
            pltpu.VMEM((2, _CQ, D), jnp.float32),
            pltpu.VMEM((2, _CQ, D), jnp.float32),
            pltpu.SemaphoreType.DMA,
            pltpu.SemaphoreType.DMA,
        ],
    )
    return scatter, combine


def _stage2_body(bexp_ref, bval_ref, fstb_ref, nxe_ref, hnx_ref, sgo_ref,
                 xs_ref, wg_any, wu_any, wd_any, ys_ref,
                 wbuf_ref, wguc_ref, wdc_ref, sem):
    b = pl.program_id(0)
    e_cur = bexp_ref[b]
    slot = lax.rem(sgo_ref[b], 2)
    nslot = 1 - slot
    valid = bval_ref[b] == 1
    first = jnp.logical_and(valid, b == fstb_ref[b])

    def copies(e, s):
        return [
            pltpu.make_async_copy(wg_any.at[e], wbuf_ref.at[s, 0], sem),
            pltpu.make_async_copy(wu_any.at[e], wbuf_ref.at[s, 1], sem),
            pltpu.make_async_copy(wd_any.at[e], wbuf_ref.at[s, 2], sem),
        ]

    @pl.when(b == 0)
    def _():
        for c in copies(e_cur, slot):
            c.start()

    @pl.when(first)
    def _():
        for c in copies(e_cur, slot):
            c.wait()

        @pl.when(hnx_ref[b] == 1)
        def _():
            for c in copies(nxe_ref[b], nslot):
                c.start()

        wguc_ref[:, :FF] = wbuf_ref[slot, 0].astype(jnp.bfloat16)
        wguc_ref[:, FF:] = wbuf_ref[slot, 1].astype(jnp.bfloat16)
        wdc_ref[...] = wbuf_ref[slot, 2].astype(jnp.bfloat16)

    @pl.when(valid)
    def _():
        xb = xs_ref[...].astype(jnp.bfloat16)
        gu = jnp.dot(xb, wguc_ref[...], preferred_element_type=jnp.float32)
        g = gu[:, :FF]
        u = gu[:, FF:]
        h = (g * jax.nn.sigmoid(g) * u).astype(jnp.bfloat16)
        y = jnp.dot(h, wdc_ref[...], preferred_element_type=jnp.float32)
        ys_ref[...] = y


def _stage2(bexp, bval, fstb, nxe, hnx, sgo, xs, Wg, Wu, Wd):
    grid_spec = pltpu.PrefetchScalarGridSpec(
        num_scalar_prefetch=6,
        grid=(NBLK,),
        in_specs=[
            pl.BlockSpec(
                (R, D),
                lambda b, be, bv, fb, nx, hn, sg:
                    (jnp.where(bv[b] == 1, b, NBLK - 1), 0)),
            pl.BlockSpec(memory_space=pl.ANY),
            pl.BlockSpec(memory_space=pl.ANY),
            pl.BlockSpec(memory_space=pl.ANY),
        ],
        out_specs=pl.BlockSpec((R, D),
                               lambda b, be, bv, fb, nx, hn, sg: (b, 0)),
        scratch_shapes=[
            pltpu.VMEM((2, 3, D, FF), jnp.float32),
            pltpu.VMEM((D, 2 * FF), jnp.bfloat16),
            pltpu.VMEM((FF, D), jnp.bfloat16),
            pltpu.SemaphoreType.DMA,
        ],
    )
    return pl.pallas_call(
        _stage2_body,
        grid_spec=grid_spec,
        out_shape=jax.ShapeDtypeStruct((NPAD, D), jnp.float32),
    )(bexp, bval, fstb, nxe, hnx, sgo, xs, Wg, Wu, Wd)


def kernel(x, gamma, W_r, Wg, Wu, Wd):
    (xnb, d0, d1, w0, w1, bexp, bval,
     fstb, nxe, hnx, sgo) = _stage1(x, gamma, W_r)
    _sc_scatter, _sc_combine = _sc_kernels()
    xs = _sc_scatter(xnb, d0, d1)
    ys = _stage2(bexp, bval, fstb, nxe, hnx, sgo, xs, Wg, Wu, Wd)
    return _sc_combine(ys, x, d0, d1, w0, w1)
